# baseline (speedup 1.0000x reference)
import functools

import jax
import jax.numpy as jnp
from jax import lax
from jax.experimental import pallas as pl
from jax.experimental.pallas import tpu as pltpu
from jax.experimental.pallas import tpu_sc as plsc

_OUT = 65
_L = 2048
_H = 128
_N = 16
_C = 32
_CB = 32
_NB = 4
_ALN = 72
_CH = 8

_mesh = plsc.VectorSubcoreMesh(core_axis_name="c", subcore_axis_name="s")


def _mask_kernel(x_ref, mask_ref):
    xb = x_ref[...]
    nz = xb != 0.0
    idx = jax.lax.broadcasted_iota(jnp.int32, xb.shape, 2)
    first = jnp.min(jnp.where(nz, idx, _L), axis=2, keepdims=True)
    last = jnp.max(jnp.where(nz, idx, -1), axis=2, keepdims=True)
    mask_ref[...] = (idx >= first) & (idx <= last)


@functools.partial(
    pl.kernel,
    out_type=jax.ShapeDtypeStruct((_N, _C, _OUT, _OUT), jnp.float32),
    mesh=_mesh,
    scratch_types=[
        pltpu.VMEM((_CH, _ALN, _ALN), jnp.float32),
        pltpu.VMEM((_CH, _OUT, _OUT), jnp.float32),
    ],
    compiler_params=pltpu.CompilerParams(use_tc_tiling_on_sc=False),
)
def _resize_sc(m_hbm, out_hbm, vmem_a, vmem_b):
    cid = lax.axis_index("c")
    sid = lax.axis_index("s")
    wid = sid * 2 + cid
    n = wid // 2
    cbase = (wid % 2) * 16

    def do_chunk(k, carry):
        cstart = cbase + k * _CH
        pltpu.sync_copy(
            m_hbm.at[n, pl.ds(cstart, _CH), : _ALN, : _ALN], vmem_a
        )

        def row_body(r, carry2):
            for ch in range(_CH):
                for c0 in (0, 16, 32, 48, 49):
                    vmem_b[ch, r, pl.ds(c0, 16)] = vmem_a[ch, r, pl.ds(c0, 16)]
            return carry2

        lax.fori_loop(0, _OUT, row_body, 0)
        pltpu.sync_copy(vmem_b, out_hbm.at[n, pl.ds(cstart, _CH)])
        return carry

    lax.fori_loop(0, 16 // _CH, do_chunk, 0)


def kernel(x, matrix):
    N, C, L = x.shape
    mask = pl.pallas_call(
        _mask_kernel,
        grid=(N // _NB, C // _CB),
        in_specs=[pl.BlockSpec((_NB, _CB, L), lambda n, c: (n, c, 0))],
        out_specs=pl.BlockSpec((_NB, _CB, L), lambda n, c: (n, c, 0)),
        out_shape=jax.ShapeDtypeStruct((N, C, L), jnp.bool_),
    )(x)
    resized = _resize_sc(matrix)
    return mask, resized


# speedup vs baseline: 1.4354x; 1.4354x over previous
import functools

import jax
import jax.numpy as jnp
from jax import lax
from jax.experimental import pallas as pl
from jax.experimental.pallas import tpu as pltpu
from jax.experimental.pallas import tpu_sc as plsc

_OUT = 65
_L = 2048
_H = 128
_N = 16
_C = 32
_CB = 32
_NB = 4
_ALN = 72
_CH = 4

_mesh = plsc.VectorSubcoreMesh(core_axis_name="c", subcore_axis_name="s")


def _mask_kernel(x_ref, mask_ref):
    xb = x_ref[...]
    nz = xb != 0.0
    idx = jax.lax.broadcasted_iota(jnp.int32, xb.shape, 2)
    first = jnp.min(jnp.where(nz, idx, _L), axis=2, keepdims=True)
    last = jnp.max(jnp.where(nz, idx, -1), axis=2, keepdims=True)
    mask_ref[...] = (idx >= first) & (idx <= last)


@functools.partial(
    pl.kernel,
    out_type=jax.ShapeDtypeStruct((_N, _C, _OUT, _OUT), jnp.float32),
    mesh=_mesh,
    scratch_types=[
        pltpu.VMEM((_CH, _ALN, _H), jnp.float32),
        pltpu.VMEM((_CH, _OUT, _OUT), jnp.float32),
    ],
)
def _resize_sc(m_hbm, out_hbm, vmem_a, vmem_b):
    cid = lax.axis_index("c")
    sid = lax.axis_index("s")
    wid = sid * 2 + cid
    n = wid // 2
    cbase = (wid % 2) * 16

    def do_chunk(k, carry):
        cstart = cbase + k * _CH
        pltpu.sync_copy(
            m_hbm.at[n, pl.ds(cstart, _CH), : _ALN, :], vmem_a
        )

        def row_body(r, carry2):
            for ch in range(_CH):
                for c0 in (0, 16, 32, 48, 49):
                    vmem_b[ch, r, pl.ds(c0, 16)] = vmem_a[ch, r, pl.ds(c0, 16)]
            return carry2

        lax.fori_loop(0, _OUT, row_body, 0)
        pltpu.sync_copy(vmem_b, out_hbm.at[n, pl.ds(cstart, _CH)])
        return carry

    lax.fori_loop(0, 16 // _CH, do_chunk, 0)


def kernel(x, matrix):
    N, C, L = x.shape
    mask = pl.pallas_call(
        _mask_kernel,
        grid=(N // _NB, C // _CB),
        in_specs=[pl.BlockSpec((_NB, _CB, L), lambda n, c: (n, c, 0))],
        out_specs=pl.BlockSpec((_NB, _CB, L), lambda n, c: (n, c, 0)),
        out_shape=jax.ShapeDtypeStruct((N, C, L), jnp.bool_),
    )(x)
    resized = _resize_sc(matrix)
    return mask, resized


# speedup vs baseline: 2.2884x; 1.5943x over previous
import jax
import jax.numpy as jnp
from jax.experimental import pallas as pl

_OUT = 65
_L = 2048
_H = 128
_MROWS = 72
_CB = 32
_NB = 8


def _fused_kernel(x_ref, m_ref, mask_ref, out_ref):
    xb = x_ref[...]
    nz = xb != 0.0
    idx = jax.lax.broadcasted_iota(jnp.int32, xb.shape, 2)
    first = jnp.min(jnp.where(nz, idx, _L), axis=2, keepdims=True)
    last = jnp.max(jnp.where(nz, idx, -1), axis=2, keepdims=True)
    mask_ref[...] = (idx >= first) & (idx <= last)
    out_ref[...] = m_ref[:, :, :_OUT, :_OUT]


def kernel(x, matrix):
    N, C, L = x.shape
    mask, resized = pl.pallas_call(
        _fused_kernel,
        grid=(N // _NB, C // _CB),
        in_specs=[
            pl.BlockSpec((_NB, _CB, L), lambda n, c: (n, c, 0)),
            pl.BlockSpec((_NB, _CB, _MROWS, _H), lambda n, c: (n, c, 0, 0)),
        ],
        out_specs=[
            pl.BlockSpec((_NB, _CB, L), lambda n, c: (n, c, 0)),
            pl.BlockSpec((_NB, _CB, _OUT, _OUT), lambda n, c: (n, c, 0, 0)),
        ],
        out_shape=[
            jax.ShapeDtypeStruct((N, C, L), jnp.bool_),
            jax.ShapeDtypeStruct((N, C, _OUT, _OUT), jnp.float32),
        ],
    )(x, matrix)
    return mask, resized
